# baseline (speedup 1.0000x reference)
import functools

import jax
import jax.numpy as jnp
from jax import lax
from jax.experimental import pallas as pl
from jax.experimental.pallas import tpu as pltpu
from jax.experimental.pallas import tpu_sc as plsc

_NUM_PART = 21
_D_IN = 128
_D_HID = 16
_B = 16384
_H_ALL = _NUM_PART * _D_HID
_BLK = 4096

_NC = 2
_NS = 16
_L = 16
_NW = _NC * _NS
_CH = _B // _NW


def _heads_body(x_ref, w1_ref, b1_ref, sel2_ref, b2_ref, out_ref):
    w1 = w1_ref[...]
    h = lax.dot_general(x_ref[...], w1, (((1,), (1,)), ((), ())),
                        preferred_element_type=jnp.float32)
    h = jnp.maximum(h + b1_ref[...], 0.0)
    o = jnp.dot(h, sel2_ref[...], preferred_element_type=jnp.float32)
    out_ref[...] = jnp.maximum(o + b2_ref[...], 0.0)


def _compute_all_heads(x, w1, b1f, sel2, b2row):
    return pl.pallas_call(
        _heads_body,
        grid=(_B // _BLK,),
        in_specs=[
            pl.BlockSpec((_BLK, _D_IN), lambda i: (i, 0)),
            pl.BlockSpec((_H_ALL, _D_IN), lambda i: (0, 0)),
            pl.BlockSpec((1, _H_ALL), lambda i: (0, 0)),
            pl.BlockSpec((_H_ALL, _NUM_PART), lambda i: (0, 0)),
            pl.BlockSpec((1, _NUM_PART), lambda i: (0, 0)),
        ],
        out_specs=pl.BlockSpec((_BLK, _NUM_PART), lambda i: (i, 0)),
        out_shape=jax.ShapeDtypeStruct((_B, _NUM_PART), jnp.float32),
    )(x, w1, b1f, sel2, b2row)


_sc_mesh = plsc.VectorSubcoreMesh(core_axis_name="c", subcore_axis_name="s")


@functools.partial(
    pl.kernel,
    mesh=_sc_mesh,
    out_type=jax.ShapeDtypeStruct((_B,), jnp.float32),
    scratch_types=[
        pltpu.VMEM((_CH,), jnp.int32),
        pltpu.VMEM((_CH, _NUM_PART), jnp.float32),
        pltpu.VMEM((_CH,), jnp.float32),
        pltpu.SemaphoreType.DMA,
    ],
    compiler_params=pltpu.CompilerParams(needs_layout_passes=False),
)
def _sc_select(oall_hbm, subj_hbm, out_hbm, subj_v, tab_v, out_v, sem):
    wid = lax.axis_index("s") * _NC + lax.axis_index("c")
    base = wid * _CH
    c1 = pltpu.async_copy(subj_hbm.at[pl.ds(base, _CH)], subj_v, sem)
    c2 = pltpu.async_copy(oall_hbm.at[pl.ds(base, _CH)], tab_v, sem)
    c1.wait()
    c2.wait()
    @plsc.parallel_loop(0, _CH, _L, unroll=8)
    def _gather_loop(off):
        subj = subj_v[pl.ds(off, _L)]
        rows = lax.iota(jnp.int32, _L) + off
        out_v[pl.ds(off, _L)] = plsc.load_gather(tab_v, [rows, subj])
    pltpu.sync_copy(out_v, out_hbm.at[pl.ds(base, _CH)])


def kernel(x, SubjId, W1, b1, W2, b2):
    w1km = W1.transpose(1, 0, 2).reshape(_H_ALL, _D_IN)
    b1f = b1.transpose(1, 0).reshape(1, _H_ALL)
    w2col = W2.transpose(1, 2, 0).reshape(_H_ALL, 1)
    sel2 = jnp.where(
        jnp.arange(_H_ALL, dtype=jnp.int32)[:, None] % _NUM_PART
        == jnp.arange(_NUM_PART, dtype=jnp.int32)[None, :],
        w2col, 0.0)
    out_all = _compute_all_heads(x, w1km, b1f, sel2, b2.reshape(1, _NUM_PART))
    out = _sc_select(out_all, SubjId)
    return out.reshape(_B, 1)


# speedup vs baseline: 1.0503x; 1.0503x over previous
import functools

import jax
import jax.numpy as jnp
from jax import lax
from jax.experimental import pallas as pl
from jax.experimental.pallas import tpu as pltpu
from jax.experimental.pallas import tpu_sc as plsc

_NUM_PART = 21
_D_IN = 128
_D_HID = 16
_B = 16384
_H_ALL = _NUM_PART * _D_HID
_BLK = 4096

_NC = 2
_NS = 16
_L = 16
_NW = _NC * _NS
_CH = _B // _NW


def _heads_body(x_ref, w1_ref, b1_ref, sel2_ref, b2_ref, out_ref):
    w1 = w1_ref[...].reshape(_H_ALL, _D_IN)
    h = lax.dot_general(x_ref[...], w1, (((1,), (1,)), ((), ())),
                        preferred_element_type=jnp.float32)
    h = jnp.maximum(h + b1_ref[...], 0.0)
    o = jnp.dot(h, sel2_ref[...], preferred_element_type=jnp.float32)
    out_ref[...] = jnp.maximum(o + b2_ref[...], 0.0)


def _compute_all_heads(x, w1, b1f, sel2, b2row):
    return pl.pallas_call(
        _heads_body,
        grid=(_B // _BLK,),
        in_specs=[
            pl.BlockSpec((_BLK, _D_IN), lambda i: (i, 0)),
            pl.BlockSpec((_NUM_PART, _D_HID, _D_IN), lambda i: (0, 0, 0)),
            pl.BlockSpec((1, _H_ALL), lambda i: (0, 0)),
            pl.BlockSpec((_H_ALL, _NUM_PART), lambda i: (0, 0)),
            pl.BlockSpec((1, _NUM_PART), lambda i: (0, 0)),
        ],
        out_specs=pl.BlockSpec((_BLK, _NUM_PART), lambda i: (i, 0)),
        out_shape=jax.ShapeDtypeStruct((_B, _NUM_PART), jnp.float32),
    )(x, w1, b1f, sel2, b2row)


_sc_mesh = plsc.VectorSubcoreMesh(core_axis_name="c", subcore_axis_name="s",
                                  num_cores=_NC)


@functools.partial(
    pl.kernel,
    mesh=_sc_mesh,
    out_type=jax.ShapeDtypeStruct((_B,), jnp.float32),
    scratch_types=[
        pltpu.VMEM((_CH,), jnp.int32),
        pltpu.VMEM((_CH, _NUM_PART), jnp.float32),
        pltpu.VMEM((_CH,), jnp.float32),
        pltpu.SemaphoreType.DMA,
    ],
    compiler_params=pltpu.CompilerParams(needs_layout_passes=False),
)
def _sc_select(oall_hbm, subj_hbm, out_hbm, subj_v, tab_v, out_v, sem):
    wid = lax.axis_index("s") * _NC + lax.axis_index("c")
    base = wid * _CH
    c1 = pltpu.async_copy(subj_hbm.at[pl.ds(base, _CH)], subj_v, sem)
    c2 = pltpu.async_copy(oall_hbm.at[pl.ds(base, _CH)], tab_v, sem)
    c1.wait()
    c2.wait()
    @plsc.parallel_loop(0, _CH, _L, unroll=8)
    def _gather_loop(off):
        subj = subj_v[pl.ds(off, _L)]
        rows = lax.iota(jnp.int32, _L) + off
        out_v[pl.ds(off, _L)] = plsc.load_gather(tab_v, [rows, subj])
    pltpu.sync_copy(out_v, out_hbm.at[pl.ds(base, _CH)])


def kernel(x, SubjId, W1, b1, W2, b2):
    b1f = b1.reshape(1, _H_ALL)
    sel2 = jnp.where(
        jnp.arange(_H_ALL, dtype=jnp.int32)[:, None] // _D_HID
        == jnp.arange(_NUM_PART, dtype=jnp.int32)[None, :],
        W2.reshape(_H_ALL, 1), 0.0)
    out_all = _compute_all_heads(x, W1, b1f, sel2, b2.reshape(1, _NUM_PART))
    out = _sc_select(out_all, SubjId)
    return out.reshape(_B, 1)
